# baseline (speedup 1.0000x reference)
import functools

import jax
import jax.numpy as jnp
from jax.experimental import pallas as pl
from jax.experimental.pallas import tpu as pltpu


def _pool_excite_kernel(x_ref, w1_ref, b1_ref, w2_ref, b2_ref, g_ref, *,
                        inv_hw, hw):
    pooled = jnp.sum(x_ref[:, :, :hw], axis=-1, dtype=jnp.float32) * inv_hw
    h = jnp.maximum(
        jax.lax.dot_general(pooled, w1_ref[...], (((1,), (1,)), ((), ())),
                            preferred_element_type=jnp.float32)
        + b1_ref[...], 0.0)
    g_ref[...] = jax.nn.sigmoid(
        jax.lax.dot_general(h, w2_ref[...], (((1,), (1,)), ((), ())),
                            preferred_element_type=jnp.float32)
        + b2_ref[...])


def _pick_images_per_block(n, bytes_per_image, budget):
    best = 1
    for d in range(1, n + 1):
        if n % d == 0 and d * bytes_per_image <= budget:
            best = d
    return best


def kernel(x_nchw, w1, b1, w2, b2):
    N, C, H, W = x_nchw.shape
    Cr = w1.shape[0]
    HW = H * W
    dtype = x_nchw.dtype

    x3 = x_nchw.reshape(N, C, HW)
    b1r = b1.reshape(1, Cr)
    b2r = b2.reshape(1, C)
    inv_hw = 1.0 / float(HW)

    lanes = ((HW + 127) // 128) * 128
    bytes_per_image = C * lanes * dtype.itemsize
    nb = _pick_images_per_block(N, bytes_per_image, budget=8 << 20)
    grid = (N // nb,)

    gates = pl.pallas_call(
        functools.partial(_pool_excite_kernel, inv_hw=inv_hw, hw=HW),
        out_shape=jax.ShapeDtypeStruct((N, C), jnp.float32),
        grid=grid,
        in_specs=[
            pl.BlockSpec((nb, C, lanes), lambda i: (i, 0, 0)),
            pl.BlockSpec((Cr, C), lambda i: (0, 0)),
            pl.BlockSpec((1, Cr), lambda i: (0, 0)),
            pl.BlockSpec((C, Cr), lambda i: (0, 0)),
            pl.BlockSpec((1, C), lambda i: (0, 0)),
        ],
        out_specs=pl.BlockSpec((nb, C), lambda i: (i, 0)),
        compiler_params=pltpu.CompilerParams(
            dimension_semantics=("arbitrary",),
            vmem_limit_bytes=48 << 20,
            disable_bounds_checks=True,
        ),
    )(x3, w1, b1r, w2, b2r)

    out3 = x3 * gates.astype(dtype)[:, :, None]
    return out3.reshape(N, C, H, W)


# speedup vs baseline: 1.0024x; 1.0024x over previous
import functools

import jax
import jax.numpy as jnp
from jax.experimental import pallas as pl
from jax.experimental.pallas import tpu as pltpu


def _pool_excite_kernel(x_ref, w1_ref, b1_ref, w2_ref, b2_ref, g_ref, *,
                        inv_hw, hw):
    pooled = jnp.sum(x_ref[:, :, :hw], axis=-1, dtype=jnp.float32) * inv_hw
    h = jnp.maximum(
        jax.lax.dot_general(pooled, w1_ref[...], (((1,), (1,)), ((), ())),
                            preferred_element_type=jnp.float32)
        + b1_ref[...], 0.0)
    g_ref[...] = jax.nn.sigmoid(
        jax.lax.dot_general(h, w2_ref[...], (((1,), (1,)), ((), ())),
                            preferred_element_type=jnp.float32)
        + b2_ref[...])


def _pick_images_per_block(n, bytes_per_image, budget):
    best = 1
    for d in range(1, n + 1):
        if n % d == 0 and d * bytes_per_image <= budget:
            best = d
    return best


def kernel(x_nchw, w1, b1, w2, b2):
    N, C, H, W = x_nchw.shape
    Cr = w1.shape[0]
    HW = H * W
    dtype = x_nchw.dtype

    x3 = x_nchw.reshape(N, C, HW)
    b1r = b1.reshape(1, Cr)
    b2r = b2.reshape(1, C)
    inv_hw = 1.0 / float(HW)

    lanes = ((HW + 127) // 128) * 128
    bytes_per_image = C * lanes * dtype.itemsize
    nb = _pick_images_per_block(N, bytes_per_image, budget=16 << 20)
    grid = (N // nb,)

    gates = pl.pallas_call(
        functools.partial(_pool_excite_kernel, inv_hw=inv_hw, hw=HW),
        out_shape=jax.ShapeDtypeStruct((N, C), jnp.float32),
        grid=grid,
        in_specs=[
            pl.BlockSpec((nb, C, lanes), lambda i: (i, 0, 0)),
            pl.BlockSpec((Cr, C), lambda i: (0, 0)),
            pl.BlockSpec((1, Cr), lambda i: (0, 0)),
            pl.BlockSpec((C, Cr), lambda i: (0, 0)),
            pl.BlockSpec((1, C), lambda i: (0, 0)),
        ],
        out_specs=pl.BlockSpec((nb, C), lambda i: (i, 0)),
        compiler_params=pltpu.CompilerParams(
            dimension_semantics=("arbitrary",),
            vmem_limit_bytes=48 << 20,
            disable_bounds_checks=True,
        ),
    )(x3, w1, b1r, w2, b2r)

    out3 = x3 * gates.astype(dtype)[:, :, None]
    return out3.reshape(N, C, H, W)
